# baseline (speedup 1.0000x reference)
import functools

import jax
import jax.numpy as jnp
from jax import lax
from jax.experimental import pallas as pl
from jax.experimental.pallas import tpu as pltpu
from jax.experimental.pallas import tpu_sc as plsc

NC = 2
NS = 16
CH = 128


@functools.partial(jax.jit, static_argnums=(4, 5, 6))
def _sc_agg(tab, rows, cols, zeros, nt, e_sc, feat):
    ept = e_sc // NS
    nch = ept // CH
    rpt = nt // NS
    mesh = plsc.VectorSubcoreMesh(
        core_axis_name="c", subcore_axis_name="s",
        num_cores=NC, num_subcores=NS)

    @functools.partial(
        pl.kernel,
        out_type=jax.ShapeDtypeStruct((NC, nt, feat), jnp.float32),
        mesh=mesh,
        scratch_types=[
            pltpu.VMEM((CH,), jnp.int32),
            pltpu.VMEM((CH,), jnp.int32),
            pltpu.VMEM((CH, feat), jnp.float32),
            pltpu.VMEM_SHARED((nt, feat), jnp.float32),
            pltpu.SemaphoreType.DMA,
        ],
    )
    def k(tab, rows, cols, zeros, out, colbuf, rowbuf, gbuf, acc, sem):
        c = lax.axis_index("c")
        s = lax.axis_index("s")

        pltpu.sync_copy(zeros, gbuf)

        @pl.loop(0, rpt // CH)
        def _(kk):
            pltpu.sync_copy(gbuf, acc.at[pl.ds(s * rpt + kk * CH, CH)])

        plsc.subcore_barrier()

        base = s * ept

        @pl.loop(0, nch)
        def _(j):
            off = base + j * CH
            pltpu.sync_copy(cols.at[c, pl.ds(off, CH)], colbuf)
            pltpu.sync_copy(rows.at[c, pl.ds(off, CH)], rowbuf)
            pltpu.async_copy(tab.at[colbuf], gbuf, sem).wait()
            pltpu.sync_copy(gbuf, acc.at[rowbuf], add=True)

        plsc.subcore_barrier()

        @pl.loop(0, rpt // CH)
        def _(kk):
            r0 = s * rpt + kk * CH
            pltpu.sync_copy(acc.at[pl.ds(r0, CH)], gbuf)
            pltpu.sync_copy(gbuf, out.at[c, pl.ds(r0, CH)])

    return k(tab, rows, cols, zeros)


@functools.partial(jax.jit, static_argnums=(2, 3))
def _sc_count(rows, ones, nt, e_sc):
    ept = e_sc // NS
    nch = ept // CH
    rpt = nt // NS
    mesh = plsc.VectorSubcoreMesh(
        core_axis_name="c", subcore_axis_name="s",
        num_cores=NC, num_subcores=NS)

    @functools.partial(
        pl.kernel,
        out_type=jax.ShapeDtypeStruct((NC, nt, 16), jnp.float32),
        mesh=mesh,
        scratch_types=[
            pltpu.VMEM((CH,), jnp.int32),
            pltpu.VMEM((CH, 16), jnp.float32),
            pltpu.VMEM((CH, 16), jnp.float32),
            pltpu.VMEM_SHARED((nt, 16), jnp.float32),
        ],
    )
    def k(rows, ones, out, rowbuf, obuf, zbuf, acc):
        c = lax.axis_index("c")
        s = lax.axis_index("s")

        pltpu.sync_copy(ones.at[pl.ds(0, CH)], obuf)
        pltpu.sync_copy(ones.at[pl.ds(CH, CH)], zbuf)

        @pl.loop(0, rpt // CH)
        def _(kk):
            pltpu.sync_copy(zbuf, acc.at[pl.ds(s * rpt + kk * CH, CH)])

        plsc.subcore_barrier()

        base = s * ept

        @pl.loop(0, nch)
        def _(j):
            off = base + j * CH
            pltpu.sync_copy(rows.at[c, pl.ds(off, CH)], rowbuf)
            pltpu.sync_copy(obuf, acc.at[rowbuf], add=True)

        plsc.subcore_barrier()

        @pl.loop(0, rpt // CH)
        def _(kk):
            r0 = s * rpt + kk * CH
            pltpu.sync_copy(acc.at[pl.ds(r0, CH)], obuf)
            pltpu.sync_copy(obuf, out.at[c, pl.ds(r0, CH)])

    return k(rows, ones)


def _rc(cr):
    return 1.0 / jnp.maximum(cr[0, :, 0:1] + cr[1, :, 0:1], 1.0)


def _g1_body(S, cr, hp, Wr, Br, z, t, rh):
    rc = _rc(cr)
    ax = S[0] * rc
    ah = S[1] * rc
    dot = lambda a, w: jnp.dot(a, w, preferred_element_type=jnp.float32)
    z[...] = jax.nn.sigmoid(dot(ax, Wr[0]) + dot(ah, Wr[1]) + (Br[0] + Br[1]))
    r = jax.nn.sigmoid(dot(ax, Wr[2]) + dot(ah, Wr[3]) + (Br[2] + Br[3]))
    t[...] = dot(ax, Wr[4]) + Br[4]
    rh[...] = r * hp[...]


def _g2_body(t, P, cr, hp, z, Wr, Br, hn):
    arh = (P[0] + P[1]) * _rc(cr)
    g = jnp.tanh(t[...] + jnp.dot(arh, Wr[5], preferred_element_type=jnp.float32)
                 + Br[5])
    zz = z[...]
    hn[...] = zz * hp[...] + (1.0 - zz) * g


@functools.partial(jax.jit, static_argnums=(5,))
def _tc_gates1(S, cr, hp, Wl, Bl, bn):
    n = hp.shape[0]
    return pl.pallas_call(
        _g1_body,
        grid=(n // bn,),
        in_specs=[
            pl.BlockSpec((2, bn, 128), lambda i: (0, i, 0)),
            pl.BlockSpec((2, bn, 16), lambda i: (0, i, 0)),
            pl.BlockSpec((bn, 128), lambda i: (i, 0)),
            pl.BlockSpec((6, 128, 128), lambda i: (0, 0, 0)),
            pl.BlockSpec((6, 128), lambda i: (0, 0)),
        ],
        out_specs=[pl.BlockSpec((bn, 128), lambda i: (i, 0))] * 3,
        out_shape=[jax.ShapeDtypeStruct((n, 128), jnp.float32)] * 3,
    )(S, cr, hp, Wl, Bl)


@functools.partial(jax.jit, static_argnums=(7,))
def _tc_gates2(t, P, cr, hp, z, Wl, Bl, bn):
    n = hp.shape[0]
    return pl.pallas_call(
        _g2_body,
        grid=(n // bn,),
        in_specs=[
            pl.BlockSpec((bn, 128), lambda i: (i, 0)),
            pl.BlockSpec((2, bn, 128), lambda i: (0, i, 0)),
            pl.BlockSpec((2, bn, 16), lambda i: (0, i, 0)),
            pl.BlockSpec((bn, 128), lambda i: (i, 0)),
            pl.BlockSpec((bn, 128), lambda i: (i, 0)),
            pl.BlockSpec((6, 128, 128), lambda i: (0, 0, 0)),
            pl.BlockSpec((6, 128), lambda i: (0, 0)),
        ],
        out_specs=pl.BlockSpec((bn, 128), lambda i: (i, 0)),
        out_shape=jax.ShapeDtypeStruct((n, 128), jnp.float32),
    )(t, P, cr, hp, z, Wl, Bl)


@jax.jit
def _run(inp, edgidx, h, W, B):
    n = inp.shape[0]
    e = edgidx.shape[1]
    nlayers = h.shape[0]
    bn = 1000 if n % 1000 == 0 else 8 * (n // 8)

    nt = ((n + 1 + NS * CH - 1) // (NS * CH)) * (NS * CH)

    row, col = edgidx[0], edgidx[1]
    mask = row != col
    rowe = jnp.where(mask, row, n)
    ar = jnp.arange(n, dtype=jnp.int32)
    row_f = jnp.concatenate([rowe, ar])
    col_f = jnp.concatenate([col, ar])
    etot = e + n
    quant = 4 * NS * CH
    ep = ((etot + quant - 1) // quant) * quant
    pad = ep - etot
    row_p = jnp.concatenate([row_f, jnp.full((pad,), n, jnp.int32)])
    col_p = jnp.concatenate([col_f, jnp.zeros((pad,), jnp.int32)])

    rows_d = jnp.stack([row_p, row_p])
    cols_d = jnp.stack([col_p, col_p + n])
    rows_s = row_p.reshape(2, ep // 2)
    cols_s = col_p.reshape(2, ep // 2)

    z128 = jnp.zeros((CH, 128), jnp.float32)
    ones_z = jnp.concatenate(
        [jnp.ones((CH, 16), jnp.float32), jnp.zeros((CH, 16), jnp.float32)])

    cr = _sc_count(rows_s, ones_z, nt, ep // 2)

    h_prev = inp
    h_out = []
    for i in range(nlayers):
        hp = h[i]
        tab = jnp.concatenate([h_prev, hp], axis=0)
        S = _sc_agg(tab, rows_d, cols_d, z128, nt, ep, 128)
        z, t, rh = _tc_gates1(S, cr, hp, W[i], B[i], bn)
        P = _sc_agg(rh, rows_s, cols_s, z128, nt, ep // 2, 128)
        hn = _tc_gates2(t, P, cr, hp, z, W[i], B[i], bn)
        h_out.append(hn)
        h_prev = hn

    out = jnp.stack(h_out, axis=0)
    return (out, out)


def kernel(inp, edgidx, h, W, B):
    return _run(inp, edgidx, h, W, B)


# speedup vs baseline: 1.3943x; 1.3943x over previous
import functools

import jax
import jax.numpy as jnp
from jax import lax
from jax.experimental import pallas as pl
from jax.experimental.pallas import tpu as pltpu
from jax.experimental.pallas import tpu_sc as plsc

NC = 2
NS = 16
CH = 128


@functools.partial(jax.jit, static_argnums=(4, 5, 6))
def _sc_agg(tab, rows, cols, zeros, nt, e_sc, feat):
    ept = e_sc // NS
    nch = ept // CH
    rpt = nt // NS
    mesh = plsc.VectorSubcoreMesh(
        core_axis_name="c", subcore_axis_name="s",
        num_cores=NC, num_subcores=NS)

    @functools.partial(
        pl.kernel,
        out_type=jax.ShapeDtypeStruct((NC, nt, feat), jnp.float32),
        mesh=mesh,
        scratch_types=[
            pltpu.VMEM((CH,), jnp.int32),
            pltpu.VMEM((CH,), jnp.int32),
            pltpu.VMEM((CH, feat), jnp.float32),
            pltpu.VMEM((CH, feat), jnp.float32),
            pltpu.VMEM_SHARED((nt, feat), jnp.float32),
            pltpu.SemaphoreType.DMA,
        ],
    )
    def k(tab, rows, cols, zeros, out, colbuf, rowbuf, gbuf, zbuf, acc, sem):
        c = lax.axis_index("c")
        s = lax.axis_index("s")

        pltpu.sync_copy(zeros, zbuf)

        @pl.loop(0, rpt // CH)
        def _(kk):
            pltpu.sync_copy(zbuf, acc.at[pl.ds(s * rpt + kk * CH, CH)])

        plsc.subcore_barrier()

        base = s * ept

        @pl.loop(0, nch)
        def _(j):
            off = base + j * CH
            pltpu.sync_copy(cols.at[c, pl.ds(off, CH)], colbuf)
            pltpu.sync_copy(rows.at[c, pl.ds(off, CH)], rowbuf)
            pltpu.async_copy(tab.at[colbuf], gbuf, sem).wait()
            pltpu.sync_copy(gbuf, acc.at[rowbuf], add=True)

        plsc.subcore_barrier()

        @pl.loop(0, rpt // CH)
        def _(kk):
            r0 = s * rpt + kk * CH
            pltpu.sync_copy(acc.at[pl.ds(r0, CH)], gbuf)
            pltpu.sync_copy(gbuf, out.at[c, pl.ds(r0, CH)])

    return k(tab, rows, cols, zeros)


@functools.partial(jax.jit, static_argnums=(2, 3))
def _sc_count(rows, ones, nt, e_sc):
    ept = e_sc // NS
    nch = ept // CH
    rpt = nt // NS
    mesh = plsc.VectorSubcoreMesh(
        core_axis_name="c", subcore_axis_name="s",
        num_cores=NC, num_subcores=NS)

    @functools.partial(
        pl.kernel,
        out_type=jax.ShapeDtypeStruct((NC, nt, 16), jnp.float32),
        mesh=mesh,
        scratch_types=[
            pltpu.VMEM((CH,), jnp.int32),
            pltpu.VMEM((CH, 16), jnp.float32),
            pltpu.VMEM((CH, 16), jnp.float32),
            pltpu.VMEM_SHARED((nt, 16), jnp.float32),
        ],
    )
    def k(rows, ones, out, rowbuf, obuf, zbuf, acc):
        c = lax.axis_index("c")
        s = lax.axis_index("s")

        pltpu.sync_copy(ones.at[pl.ds(0, CH)], obuf)
        pltpu.sync_copy(ones.at[pl.ds(CH, CH)], zbuf)

        @pl.loop(0, rpt // CH)
        def _(kk):
            pltpu.sync_copy(zbuf, acc.at[pl.ds(s * rpt + kk * CH, CH)])

        plsc.subcore_barrier()

        base = s * ept

        @pl.loop(0, nch)
        def _(j):
            off = base + j * CH
            pltpu.sync_copy(rows.at[c, pl.ds(off, CH)], rowbuf)
            pltpu.sync_copy(obuf, acc.at[rowbuf], add=True)

        plsc.subcore_barrier()

        @pl.loop(0, rpt // CH)
        def _(kk):
            r0 = s * rpt + kk * CH
            pltpu.sync_copy(acc.at[pl.ds(r0, CH)], obuf)
            pltpu.sync_copy(obuf, out.at[c, pl.ds(r0, CH)])

    return k(rows, ones)


def _rc(cr):
    return 1.0 / jnp.maximum(cr[0, :, 0:1] + cr[1, :, 0:1], 1.0)


def _g1_body(S, cr, hp, Wr, Br, z, t, rh):
    rc = _rc(cr)
    ax = S[0] * rc
    ah = S[1] * rc
    dot = lambda a, w: jnp.dot(a, w, preferred_element_type=jnp.float32)
    z[...] = jax.nn.sigmoid(dot(ax, Wr[0]) + dot(ah, Wr[1]) + (Br[0] + Br[1]))
    r = jax.nn.sigmoid(dot(ax, Wr[2]) + dot(ah, Wr[3]) + (Br[2] + Br[3]))
    t[...] = dot(ax, Wr[4]) + Br[4]
    rh[...] = r * hp[...]


def _g2_body(t, P, cr, hp, z, Wr, Br, hn):
    arh = (P[0] + P[1]) * _rc(cr)
    g = jnp.tanh(t[...] + jnp.dot(arh, Wr[5], preferred_element_type=jnp.float32)
                 + Br[5])
    zz = z[...]
    hn[...] = zz * hp[...] + (1.0 - zz) * g


@functools.partial(jax.jit, static_argnums=(5,))
def _tc_gates1(S, cr, hp, Wl, Bl, bn):
    n = hp.shape[0]
    return pl.pallas_call(
        _g1_body,
        grid=(n // bn,),
        in_specs=[
            pl.BlockSpec((2, bn, 128), lambda i: (0, i, 0)),
            pl.BlockSpec((2, bn, 16), lambda i: (0, i, 0)),
            pl.BlockSpec((bn, 128), lambda i: (i, 0)),
            pl.BlockSpec((6, 128, 128), lambda i: (0, 0, 0)),
            pl.BlockSpec((6, 128), lambda i: (0, 0)),
        ],
        out_specs=[pl.BlockSpec((bn, 128), lambda i: (i, 0))] * 3,
        out_shape=[jax.ShapeDtypeStruct((n, 128), jnp.float32)] * 3,
    )(S, cr, hp, Wl, Bl)


@functools.partial(jax.jit, static_argnums=(7,))
def _tc_gates2(t, P, cr, hp, z, Wl, Bl, bn):
    n = hp.shape[0]
    return pl.pallas_call(
        _g2_body,
        grid=(n // bn,),
        in_specs=[
            pl.BlockSpec((bn, 128), lambda i: (i, 0)),
            pl.BlockSpec((2, bn, 128), lambda i: (0, i, 0)),
            pl.BlockSpec((2, bn, 16), lambda i: (0, i, 0)),
            pl.BlockSpec((bn, 128), lambda i: (i, 0)),
            pl.BlockSpec((bn, 128), lambda i: (i, 0)),
            pl.BlockSpec((6, 128, 128), lambda i: (0, 0, 0)),
            pl.BlockSpec((6, 128), lambda i: (0, 0)),
        ],
        out_specs=pl.BlockSpec((bn, 128), lambda i: (i, 0)),
        out_shape=jax.ShapeDtypeStruct((n, 128), jnp.float32),
    )(t, P, cr, hp, z, Wl, Bl)


@jax.jit
def _run(inp, edgidx, h, W, B):
    n = inp.shape[0]
    e = edgidx.shape[1]
    nlayers = h.shape[0]
    bn = 1000 if n % 1000 == 0 else 8 * (n // 8)

    nt = ((n + 1 + NS * CH - 1) // (NS * CH)) * (NS * CH)

    row, col = edgidx[0], edgidx[1]
    mask = row != col
    rowe = jnp.where(mask, row, n)
    ar = jnp.arange(n, dtype=jnp.int32)
    row_f = jnp.concatenate([rowe, ar])
    col_f = jnp.concatenate([col, ar])
    etot = e + n
    ep = ((etot + 2 * NS * CH - 1) // (2 * NS * CH)) * (2 * NS * CH)
    pad = ep - etot
    row_p = jnp.concatenate([row_f, jnp.full((pad,), n, jnp.int32)])
    col_p = jnp.concatenate([col_f, jnp.zeros((pad,), jnp.int32)])

    rows_d = jnp.stack([row_p, row_p])
    cols_d = jnp.stack([col_p, col_p + n])
    rows_s = row_p.reshape(2, ep // 2)
    cols_s = col_p.reshape(2, ep // 2)

    z128 = jnp.zeros((CH, 128), jnp.float32)
    ones_z = jnp.concatenate(
        [jnp.ones((CH, 16), jnp.float32), jnp.zeros((CH, 16), jnp.float32)])

    cr = _sc_count(rows_s, ones_z, nt, ep // 2)

    h_prev = inp
    h_out = []
    for i in range(nlayers):
        hp = h[i]
        tab = jnp.concatenate([h_prev, hp], axis=0)
        S = _sc_agg(tab, rows_d, cols_d, z128, nt, ep, 128)
        z, t, rh = _tc_gates1(S, cr, hp, W[i], B[i], bn)
        P = _sc_agg(rh, rows_s, cols_s, z128, nt, ep // 2, 128)
        hn = _tc_gates2(t, P, cr, hp, z, W[i], B[i], bn)
        h_out.append(hn)
        h_prev = hn

    out = jnp.stack(h_out, axis=0)
    return (out, out)


def kernel(inp, edgidx, h, W, B):
    return _run(inp, edgidx, h, W, B)
